# baseline (speedup 1.0000x reference)
import functools

import jax
import jax.numpy as jnp
from jax.experimental import pallas as pl
from jax.experimental.pallas import tpu as pltpu

NUM_EXPERTS = 16
TOP_K = 2
TOKENS = 16384
D_MODEL = 2048

BLOCK_T = 2048


def _router_kernel(x_ref, wt_ref, w_out_ref, i_out_ref, ew_ref):
    x = x_ref[...]
    wt = wt_ref[...]
    logits = jnp.dot(x, wt, preferred_element_type=jnp.float32)
    lt = logits.T

    sub = jax.lax.broadcasted_iota(jnp.int32, lt.shape, 0)

    l0 = jnp.max(lt, axis=0, keepdims=True)
    i0 = jnp.argmax(lt, axis=0, keepdims=True)
    masked = jnp.where(sub == i0, -jnp.inf, lt)
    l1 = jnp.max(masked, axis=0, keepdims=True)
    i1 = jnp.argmax(masked, axis=0, keepdims=True)

    e1 = jnp.exp(l1 - l0)
    wn0 = 1.0 / (1.0 + e1)
    wn1 = e1 / (1.0 + e1)

    w_out_ref[...] = jnp.concatenate([wn0, wn1], axis=0)
    i_out_ref[...] = jnp.concatenate([i0, i1], axis=0).astype(jnp.int32)

    ew_ref[...] = wn0 * (sub == i0).astype(jnp.float32) + wn1 * (
        sub == i1
    ).astype(jnp.float32)


@jax.jit
def kernel(inputs, W):
    T, D = inputs.shape
    E = W.shape[0]
    wt = W.T
    grid = (T // BLOCK_T,)
    w_out, i_out, ew = pl.pallas_call(
        _router_kernel,
        grid=grid,
        in_specs=[
            pl.BlockSpec((BLOCK_T, D), lambda i: (i, 0)),
            pl.BlockSpec((D, E), lambda i: (0, 0)),
        ],
        out_specs=[
            pl.BlockSpec((TOP_K, BLOCK_T), lambda i: (0, i)),
            pl.BlockSpec((TOP_K, BLOCK_T), lambda i: (0, i)),
            pl.BlockSpec((E, BLOCK_T), lambda i: (0, i)),
        ],
        out_shape=[
            jax.ShapeDtypeStruct((TOP_K, T), jnp.float32),
            jax.ShapeDtypeStruct((TOP_K, T), jnp.int32),
            jax.ShapeDtypeStruct((E, T), jnp.float32),
        ],
        compiler_params=pltpu.CompilerParams(
            dimension_semantics=("arbitrary",),
        ),
    )(inputs.astype(jnp.float32), wt)
    return w_out.T, i_out.T, ew


# speedup vs baseline: 1.0514x; 1.0514x over previous
import functools

import jax
import jax.numpy as jnp
from jax.experimental import pallas as pl
from jax.experimental.pallas import tpu as pltpu

NUM_EXPERTS = 16
TOP_K = 2
TOKENS = 16384
D_MODEL = 2048

BLOCK_T = 1024


def _router_kernel(x_ref, wt_ref, w_out_ref, i_out_ref, ew_ref):
    x = x_ref[...]
    wt = wt_ref[...]
    logits = jnp.dot(x, wt, preferred_element_type=jnp.float32)
    lt = logits.T

    sub = jax.lax.broadcasted_iota(jnp.int32, lt.shape, 0)

    l0 = jnp.max(lt, axis=0, keepdims=True)
    i0 = jnp.argmax(lt, axis=0, keepdims=True)
    masked = jnp.where(sub == i0, -jnp.inf, lt)
    l1 = jnp.max(masked, axis=0, keepdims=True)
    i1 = jnp.argmax(masked, axis=0, keepdims=True)

    e1 = jnp.exp(l1 - l0)
    wn0 = 1.0 / (1.0 + e1)
    wn1 = e1 / (1.0 + e1)

    w_out_ref[...] = jnp.concatenate([wn0, wn1], axis=0)
    i_out_ref[...] = jnp.concatenate([i0, i1], axis=0).astype(jnp.int32)

    ew_ref[...] = wn0 * (sub == i0).astype(jnp.float32) + wn1 * (
        sub == i1
    ).astype(jnp.float32)


@jax.jit
def kernel(inputs, W):
    T, D = inputs.shape
    E = W.shape[0]
    wt = W.T
    grid = (T // BLOCK_T,)
    w_out, i_out, ew = pl.pallas_call(
        _router_kernel,
        grid=grid,
        in_specs=[
            pl.BlockSpec((BLOCK_T, D), lambda i: (i, 0)),
            pl.BlockSpec((D, E), lambda i: (0, 0)),
        ],
        out_specs=[
            pl.BlockSpec((TOP_K, BLOCK_T), lambda i: (0, i)),
            pl.BlockSpec((TOP_K, BLOCK_T), lambda i: (0, i)),
            pl.BlockSpec((E, BLOCK_T), lambda i: (0, i)),
        ],
        out_shape=[
            jax.ShapeDtypeStruct((TOP_K, T), jnp.float32),
            jax.ShapeDtypeStruct((TOP_K, T), jnp.int32),
            jax.ShapeDtypeStruct((E, T), jnp.float32),
        ],
        compiler_params=pltpu.CompilerParams(
            dimension_semantics=("arbitrary",),
        ),
    )(inputs.astype(jnp.float32), wt)
    return w_out.T, i_out.T, ew
